# baseline (speedup 1.0000x reference)
import math

import jax
import jax.numpy as jnp
from jax import lax
from jax.experimental import pallas as pl
from jax.experimental.pallas import tpu as pltpu
from jax.experimental.pallas import tpu_sc as plsc

N_CLUSTER = 1024
N_FEATURES = 256
BATCH = 16384
BLOCK_B = 1024

NC = 2
NS = 16
NW = NC * NS
ROWS_PER_W = BATCH // NW
CHUNK_ROWS = 64
CHUNK_WORDS = CHUNK_ROWS * N_CLUSTER
N_CHUNKS = ROWS_PER_W // CHUNK_ROWS
IDX_ROWS = 4
IDX_COLS = 128


def _tc_kernel(z_ref, lsT_ref, muT_ref, pi_ref, yc_ref, off_ref,
               at_ref, bt_ref, r_ref):
    i = pl.program_id(0)

    @pl.when(i == 0)
    def _prologue():
        lsT = lsT_ref[...]
        muT = muT_ref[...]
        inv_sT = jnp.exp(-lsT)
        at_ref[...] = -0.5 * inv_sT
        bt_ref[...] = muT * inv_sT
        const = jnp.sum(lsT, axis=0, keepdims=True)
        c = jnp.sum(muT * muT * inv_sT, axis=0, keepdims=True)
        logpi = jnp.log(pi_ref[...])
        r_ref[...] = logpi - 0.5 * (const + c
                                    + N_FEATURES * math.log(2.0 * math.pi))

    z = z_ref[...]
    zz = z * z
    logits = (r_ref[...]
              + jnp.dot(zz, at_ref[...], preferred_element_type=jnp.float32)
              + jnp.dot(z, bt_ref[...], preferred_element_type=jnp.float32))
    yita = jnp.exp(logits) + 1e-10
    s = jnp.sum(yita, axis=1, keepdims=True)
    yc = yita * (1.0 / s)
    yc_ref[...] = yc

    m = jnp.max(yc, axis=1, keepdims=True)
    iota = jax.lax.broadcasted_iota(jnp.int32, yc.shape, 1)
    idx = jnp.min(jnp.where(yc == m, iota, N_CLUSTER), axis=1, keepdims=True)
    row = jax.lax.broadcasted_iota(jnp.int32, idx.shape, 0) + i * BLOCK_B
    off_ref[...] = idx + row * N_CLUSTER


def _sc_zero_body(out_hbm, buf, sem):
    wid = lax.axis_index("s") * NC + lax.axis_index("c")
    zeros16 = jnp.zeros((16,), jnp.float32)

    def _fill(j, carry):
        for u in range(16):
            buf[pl.ds((j * 16 + u) * 16, 16)] = zeros16
        return carry

    lax.fori_loop(0, CHUNK_WORDS // 256, _fill, 0)

    base = wid * ROWS_PER_W * N_CLUSTER

    def _flush(cnk, carry):
        pltpu.async_copy(buf, out_hbm.at[pl.ds(base + cnk * CHUNK_WORDS,
                                               CHUNK_WORDS)], sem).wait()
        return carry

    lax.fori_loop(0, N_CHUNKS, _flush, 0)


def _sc_scatter_body(off_hbm, oh_hbm, idx_v, ones_v, sem):
    wid = lax.axis_index("s") * NC + lax.axis_index("c")
    base = wid * (ROWS_PER_W // IDX_COLS)
    pltpu.sync_copy(off_hbm.at[pl.ds(base, IDX_ROWS)], idx_v)
    ones16 = jnp.ones((16,), jnp.float32)
    for u in range(IDX_COLS // 16):
        ones_v[pl.ds(u * 16, 16)] = ones16
    for j in range(IDX_ROWS):
        pltpu.async_copy(ones_v, oh_hbm.at[idx_v.at[j]], sem).wait()


_sc_mesh = plsc.VectorSubcoreMesh(core_axis_name="c", subcore_axis_name="s")

_sc_zero = pl.kernel(
    _sc_zero_body,
    out_type=jax.ShapeDtypeStruct((BATCH * N_CLUSTER,), jnp.float32),
    mesh=_sc_mesh,
    scratch_types=[
        pltpu.VMEM((CHUNK_WORDS,), jnp.float32),
        pltpu.SemaphoreType.DMA,
    ],
)

_sc_scatter = pl.kernel(
    _sc_scatter_body,
    out_type=(),
    mesh=_sc_mesh,
    scratch_types=[
        pltpu.VMEM((IDX_ROWS, IDX_COLS), jnp.int32),
        pltpu.VMEM((IDX_COLS,), jnp.float32),
        pltpu.SemaphoreType.DMA,
    ],
)


@jax.jit
def kernel(z, pi_, mu_c, log_sigma2_c):
    B, d = z.shape
    K = mu_c.shape[0]
    grid = (B // BLOCK_B,)
    lsT = log_sigma2_c.T
    muT = mu_c.T
    pi2 = pi_.reshape(1, K)

    oh_flat = _sc_zero()

    yc, off = pl.pallas_call(
        _tc_kernel,
        grid=grid,
        in_specs=[
            pl.BlockSpec((BLOCK_B, d), lambda i: (i, 0)),
            pl.BlockSpec((d, K), lambda i: (0, 0)),
            pl.BlockSpec((d, K), lambda i: (0, 0)),
            pl.BlockSpec((1, K), lambda i: (0, 0)),
        ],
        out_specs=[
            pl.BlockSpec((BLOCK_B, K), lambda i: (i, 0)),
            pl.BlockSpec((BLOCK_B, 1), lambda i: (i, 0)),
        ],
        out_shape=[
            jax.ShapeDtypeStruct((B, K), jnp.float32),
            jax.ShapeDtypeStruct((B, 1), jnp.int32),
        ],
        scratch_shapes=[
            pltpu.VMEM((d, K), jnp.float32),
            pltpu.VMEM((d, K), jnp.float32),
            pltpu.VMEM((1, K), jnp.float32),
        ],
        compiler_params=pltpu.CompilerParams(
            dimension_semantics=("arbitrary",),
        ),
    )(z, lsT, muT, pi2)

    off2d = off.reshape(B // IDX_COLS, IDX_COLS)
    oh_ref = jax.new_ref(oh_flat)
    _sc_scatter(off2d, oh_ref)
    oh = oh_ref[...].reshape(B, K)
    return (yc, oh)


# speedup vs baseline: 2.8587x; 2.8587x over previous
import math

import jax
import jax.numpy as jnp
from jax.experimental import pallas as pl
from jax.experimental.pallas import tpu as pltpu

N_CLUSTER = 1024
N_FEATURES = 256
BLOCK_B = 1024


def _gmm_kernel(z_ref, lsT_ref, muT_ref, pi_ref, yc_ref, oh_ref,
                at_ref, bt_ref, r_ref, ones_ref):
    i = pl.program_id(0)

    @pl.when(i == 0)
    def _prologue():
        lsT = lsT_ref[...]
        muT = muT_ref[...]
        inv_sT = jnp.exp(-lsT)
        log2e = 1.4426950408889634
        at_ref[...] = (-0.5 * log2e) * inv_sT
        bt_ref[...] = log2e * (muT * inv_sT)
        const = jnp.sum(lsT, axis=0, keepdims=True)
        c = jnp.sum(muT * muT * inv_sT, axis=0, keepdims=True)
        logpi = jnp.log(pi_ref[...])
        r_ref[...] = log2e * (logpi - 0.5 * (const + c
                                    + N_FEATURES * math.log(2.0 * math.pi)))
        ones_ref[...] = jnp.ones_like(ones_ref)

    z = z_ref[...]
    zz = z * z
    logits = (r_ref[...]
              + jnp.dot(zz, at_ref[...], preferred_element_type=jnp.float32)
              + jnp.dot(z, bt_ref[...], preferred_element_type=jnp.float32))
    yita = jnp.exp2(logits) + 1e-10
    s = jnp.sum(yita, axis=1, keepdims=True)
    yc = yita * (1.0 / s)
    yc_ref[...] = yc

    m = jnp.max(yc, axis=1, keepdims=True)
    iota = jax.lax.broadcasted_iota(jnp.int32, yc.shape, 1)
    idx = jnp.min(jnp.where(yc == m, iota, N_CLUSTER), axis=1, keepdims=True)
    oh_ref[...] = (iota == idx).astype(jnp.float32)


@jax.jit
def kernel(z, pi_, mu_c, log_sigma2_c):
    B, d = z.shape
    K = mu_c.shape[0]
    grid = (B // BLOCK_B,)
    lsT = log_sigma2_c.T
    muT = mu_c.T
    pi2 = pi_.reshape(1, K)

    yc, oh = pl.pallas_call(
        _gmm_kernel,
        grid=grid,
        in_specs=[
            pl.BlockSpec((BLOCK_B, d), lambda i: (i, 0)),
            pl.BlockSpec((d, K), lambda i: (0, 0)),
            pl.BlockSpec((d, K), lambda i: (0, 0)),
            pl.BlockSpec((1, K), lambda i: (0, 0)),
        ],
        out_specs=[
            pl.BlockSpec((BLOCK_B, K), lambda i: (i, 0)),
            pl.BlockSpec((BLOCK_B, K), lambda i: (i, 0)),
        ],
        out_shape=[
            jax.ShapeDtypeStruct((B, K), jnp.float32),
            jax.ShapeDtypeStruct((B, K), jnp.float32),
        ],
        scratch_shapes=[
            pltpu.VMEM((d, K), jnp.float32),
            pltpu.VMEM((d, K), jnp.float32),
            pltpu.VMEM((1, K), jnp.float32),
            pltpu.VMEM((K, 1), jnp.float32),
        ],
        compiler_params=pltpu.CompilerParams(
            dimension_semantics=("arbitrary",),
        ),
    )(z, lsT, muT, pi2)
    return (yc, oh)
